# Initial kernel scaffold; baseline (speedup 1.0000x reference)
import jax
import jax.numpy as jnp
from jax.experimental import pallas as pl


def kernel(x, edge_index, edge_attr, batch, W_e1, b_e1, W_n1, b_n1, W_e2, b_e2, W_n2, b_n2, W_out, b_out):
    raise NotImplementedError("write your pallas kernel here")


# speedup vs baseline: 2.4753x; 2.4753x over previous
import functools

import jax
import jax.numpy as jnp
from jax import lax
from jax.experimental import pallas as pl
from jax.experimental.pallas import tpu as pltpu
from jax.experimental.pallas import tpu_sc as plsc

N = 10000
E = 320000
D = 128
DE = 16

NC = 2
NS = 16
NW = NC * NS
EPT = E // NW
CH = 80
NCH = EPT // CH
NP = 10240
RPT = NP // NS
ZR = 128
LANES = 16
DCH = D // LANES


def _sc_layer_body(h_hbm, src_hbm, dst_hbm, emb_hbm, out_hbm,
                   srcrow, dstrow, rows, embbuf, zrows, aggr, gsem):
    c = lax.axis_index("c")
    s = lax.axis_index("s")
    wid = c * NS + s
    ebase = wid * EPT

    def zero_body(r, _):
        for j in range(DCH):
            zrows[r, pl.ds(j * LANES, LANES)] = jnp.zeros((LANES,), jnp.float32)
        return 0
    lax.fori_loop(0, ZR, zero_body, 0)
    for z in range(RPT // ZR):
        pltpu.sync_copy(zrows, aggr.at[pl.ds(s * RPT + z * ZR, ZR)])
    plsc.subcore_barrier()

    def chunk_body(i, _):
        eoff = ebase + i * CH
        pltpu.sync_copy(src_hbm.at[pl.ds(eoff, CH)], srcrow)
        pltpu.sync_copy(dst_hbm.at[pl.ds(eoff, CH)], dstrow)
        pltpu.async_copy(h_hbm.at[srcrow], rows, gsem).wait()
        pltpu.sync_copy(emb_hbm.at[pl.ds(eoff, CH)], embbuf)

        def row_body(r, _):
            for j in range(DCH):
                sl = pl.ds(j * LANES, LANES)
                v = rows[r, sl] + embbuf[r, sl]
                rows[r, sl] = jnp.maximum(v, jnp.zeros((LANES,), jnp.float32))
            return 0
        lax.fori_loop(0, CH, row_body, 0)

        pltpu.sync_copy(rows, aggr.at[dstrow], add=True)
        return 0

    lax.fori_loop(0, NCH, chunk_body, 0)

    plsc.subcore_barrier()
    pltpu.sync_copy(aggr.at[pl.ds(s * RPT, RPT)],
                    out_hbm.at[c, pl.ds(s * RPT, RPT)])


_sc_layer = functools.partial(
    pl.kernel,
    out_type=jax.ShapeDtypeStruct((NC, NP, D), jnp.float32),
    mesh=plsc.VectorSubcoreMesh(core_axis_name="c", subcore_axis_name="s"),
    scratch_types=[
        pltpu.VMEM((CH,), jnp.int32),
        pltpu.VMEM((CH,), jnp.int32),
        pltpu.VMEM((CH, D), jnp.float32),
        pltpu.VMEM((CH, D), jnp.float32),
        pltpu.VMEM((ZR, D), jnp.float32),
        pltpu.VMEM_SHARED((NP, D), jnp.float32),
        pltpu.SemaphoreType.DMA,
    ],
)(_sc_layer_body)


BE = 4000
BN = 2000


def _emb_body(ea_ref, we1_ref, be1_ref, we2_ref, be2_ref, e1_ref, e2_ref):
    ea = ea_ref[...]
    e1_ref[...] = jnp.dot(ea, we1_ref[...].T,
                          preferred_element_type=jnp.float32) + be1_ref[...]
    e2_ref[...] = jnp.dot(ea, we2_ref[...].T,
                          preferred_element_type=jnp.float32) + be2_ref[...]


def _edge_embeddings(edge_attr, W_e1, b_e1, W_e2, b_e2):
    grid = (E // BE,)
    return pl.pallas_call(
        _emb_body,
        grid=grid,
        in_specs=[
            pl.BlockSpec((BE, DE), lambda i: (i, 0)),
            pl.BlockSpec((D, DE), lambda i: (0, 0)),
            pl.BlockSpec((D,), lambda i: (0,)),
            pl.BlockSpec((D, DE), lambda i: (0, 0)),
            pl.BlockSpec((D,), lambda i: (0,)),
        ],
        out_specs=[
            pl.BlockSpec((BE, D), lambda i: (i, 0)),
            pl.BlockSpec((BE, D), lambda i: (i, 0)),
        ],
        out_shape=[
            jax.ShapeDtypeStruct((E, D), jnp.float32),
            jax.ShapeDtypeStruct((E, D), jnp.float32),
        ],
    )(edge_attr, W_e1, b_e1, W_e2, b_e2)


def _update_body(h_ref, p_ref, wn_ref, bn_ref, o_ref):
    tot = h_ref[...] + p_ref[0] + p_ref[1]
    o_ref[...] = jax.nn.relu(
        jnp.dot(tot, wn_ref[...].T, preferred_element_type=jnp.float32)
        + bn_ref[...])


def _node_update(h, p, W_n, b_n):
    grid = (N // BN,)
    return pl.pallas_call(
        _update_body,
        grid=grid,
        in_specs=[
            pl.BlockSpec((BN, D), lambda i: (i, 0)),
            pl.BlockSpec((NC, BN, D), lambda i: (0, i, 0)),
            pl.BlockSpec((D, D), lambda i: (0, 0)),
            pl.BlockSpec((D,), lambda i: (0,)),
        ],
        out_specs=pl.BlockSpec((BN, D), lambda i: (i, 0)),
        out_shape=jax.ShapeDtypeStruct((N, D), jnp.float32),
    )(h, p, W_n, b_n)


def _final_body(h_ref, p_ref, wn_ref, bn_ref, wo_ref, bo_ref, o_ref):
    tot = h_ref[...] + p_ref[0] + p_ref[1]
    h3 = jax.nn.relu(
        jnp.dot(tot, wn_ref[...].T, preferred_element_type=jnp.float32)
        + bn_ref[...])
    logit = jnp.sum(h3 * wo_ref[...], axis=1, keepdims=True)
    o_ref[...] = jax.nn.sigmoid(logit + bo_ref[0])


def _final_head(h, p, W_n, b_n, W_out, b_out):
    grid = (N // BN,)
    return pl.pallas_call(
        _final_body,
        grid=grid,
        in_specs=[
            pl.BlockSpec((BN, D), lambda i: (i, 0)),
            pl.BlockSpec((NC, BN, D), lambda i: (0, i, 0)),
            pl.BlockSpec((D, D), lambda i: (0, 0)),
            pl.BlockSpec((D,), lambda i: (0,)),
            pl.BlockSpec((1, D), lambda i: (0, 0)),
            pl.BlockSpec((1,), lambda i: (0,)),
        ],
        out_specs=pl.BlockSpec((BN, 1), lambda i: (i, 0)),
        out_shape=jax.ShapeDtypeStruct((N, 1), jnp.float32),
    )(h, p, W_n, b_n, W_out, b_out)


def kernel(x, edge_index, edge_attr, batch, W_e1, b_e1, W_n1, b_n1,
           W_e2, b_e2, W_n2, b_n2, W_out, b_out):
    src = edge_index[0]
    dst = edge_index[1]

    emb1, emb2 = _edge_embeddings(edge_attr, W_e1, b_e1, W_e2, b_e2)

    p1 = _sc_layer(x, src, dst, emb1)
    h2 = _node_update(x, p1, W_n1, b_n1)
    p2 = _sc_layer(h2, src, dst, emb2)
    return _final_head(h2, p2, W_n2, b_n2, W_out, b_out)


# speedup vs baseline: 3.0852x; 1.2464x over previous
import functools

import jax
import jax.numpy as jnp
from jax import lax
from jax.experimental import pallas as pl
from jax.experimental.pallas import tpu as pltpu
from jax.experimental.pallas import tpu_sc as plsc

N = 10000
E = 320000
D = 128
DE = 16

NC = 2
NS = 16
NW = NC * NS
EPT = E // NW
CH = 40
NCH = EPT // CH
NP = 10240
RPT = NP // NS
ZR = 128
LANES = 16
DCH = D // LANES


NCHP = ((NCH + 7) // 8) * 8


def _sc_layer_body(h_hbm, src_hbm, dst_hbm, emb_hbm, out_hbm,
                   src0, src1, dst0, dst1, rows0, rows1, emb0, emb1,
                   zrows, aggr,
                   gsem0, gsem1, esem0, esem1, ssem0, ssem1, dsem0, dsem1):
    c = lax.axis_index("c")
    s = lax.axis_index("s")
    wid = c * NS + s
    ebase = wid * EPT

    def zero_body(r, _):
        for j in range(DCH):
            zrows[r, pl.ds(j * LANES, LANES)] = jnp.zeros((LANES,), jnp.float32)
        return 0
    lax.fori_loop(0, ZR, zero_body, 0)
    for z in range(RPT // ZR):
        pltpu.sync_copy(zrows, aggr.at[pl.ds(s * RPT + z * ZR, ZR)])

    plsc.subcore_barrier()

    def compute_msgs(rows_c, emb_c):
        def row_body(r, _):
            for j in range(DCH):
                sl = pl.ds(j * LANES, LANES)
                v = rows_c[r, sl] + emb_c[r, sl]
                rows_c[r, sl] = jnp.maximum(v, jnp.zeros((LANES,),
                                                         jnp.float32))
            return 0
        lax.fori_loop(0, CH, row_body, 0)

    def pair_body(k, _):
        i0 = 2 * k
        i1 = i0 + 1
        hs0 = pltpu.async_copy(src_hbm.at[wid, i0], src0, ssem0)
        hd0 = pltpu.async_copy(dst_hbm.at[wid, i0], dst0, dsem0)
        hs1 = pltpu.async_copy(src_hbm.at[wid, i1], src1, ssem1)
        hd1 = pltpu.async_copy(dst_hbm.at[wid, i1], dst1, dsem1)
        hs0.wait()
        hg0 = pltpu.async_copy(h_hbm.at[src0], rows0, gsem0)
        he0 = pltpu.async_copy(
            emb_hbm.at[pl.ds(ebase + i0 * CH, CH)], emb0, esem0)
        hs1.wait()
        hg0.wait()
        he0.wait()
        hg1 = pltpu.async_copy(h_hbm.at[src1], rows1, gsem1)
        he1 = pltpu.async_copy(
            emb_hbm.at[pl.ds(ebase + i1 * CH, CH)], emb1, esem1)
        compute_msgs(rows0, emb0)
        hd0.wait()
        pltpu.sync_copy(rows0, aggr.at[dst0], add=True)
        hg1.wait()
        he1.wait()
        compute_msgs(rows1, emb1)
        hd1.wait()
        pltpu.sync_copy(rows1, aggr.at[dst1], add=True)
        return 0

    lax.fori_loop(0, NCH // 2, pair_body, 0)

    plsc.subcore_barrier()
    pltpu.sync_copy(aggr.at[pl.ds(s * RPT, RPT)],
                    out_hbm.at[c, pl.ds(s * RPT, RPT)])


_sc_layer = functools.partial(
    pl.kernel,
    out_type=jax.ShapeDtypeStruct((NC, NP, D), jnp.float32),
    mesh=plsc.VectorSubcoreMesh(core_axis_name="c", subcore_axis_name="s"),
    scratch_types=[
        pltpu.VMEM((CH,), jnp.int32),
        pltpu.VMEM((CH,), jnp.int32),
        pltpu.VMEM((CH,), jnp.int32),
        pltpu.VMEM((CH,), jnp.int32),
        pltpu.VMEM((CH, D), jnp.float32),
        pltpu.VMEM((CH, D), jnp.float32),
        pltpu.VMEM((CH, D), jnp.float32),
        pltpu.VMEM((CH, D), jnp.float32),
        pltpu.VMEM((ZR, D), jnp.float32),
        pltpu.VMEM_SHARED((NP, D), jnp.float32),
        pltpu.SemaphoreType.DMA,
        pltpu.SemaphoreType.DMA,
        pltpu.SemaphoreType.DMA,
        pltpu.SemaphoreType.DMA,
        pltpu.SemaphoreType.DMA,
        pltpu.SemaphoreType.DMA,
        pltpu.SemaphoreType.DMA,
        pltpu.SemaphoreType.DMA,
    ],
)(_sc_layer_body)


BE = 4000
BN = 2000


def _emb_body(ea_ref, we1_ref, be1_ref, we2_ref, be2_ref, e1_ref, e2_ref):
    ea = ea_ref[...]
    e1_ref[...] = jnp.dot(ea, we1_ref[...].T,
                          preferred_element_type=jnp.float32) + be1_ref[...]
    e2_ref[...] = jnp.dot(ea, we2_ref[...].T,
                          preferred_element_type=jnp.float32) + be2_ref[...]


def _edge_embeddings(edge_attr, W_e1, b_e1, W_e2, b_e2):
    grid = (E // BE,)
    return pl.pallas_call(
        _emb_body,
        grid=grid,
        in_specs=[
            pl.BlockSpec((BE, DE), lambda i: (i, 0)),
            pl.BlockSpec((D, DE), lambda i: (0, 0)),
            pl.BlockSpec((D,), lambda i: (0,)),
            pl.BlockSpec((D, DE), lambda i: (0, 0)),
            pl.BlockSpec((D,), lambda i: (0,)),
        ],
        out_specs=[
            pl.BlockSpec((BE, D), lambda i: (i, 0)),
            pl.BlockSpec((BE, D), lambda i: (i, 0)),
        ],
        out_shape=[
            jax.ShapeDtypeStruct((E, D), jnp.float32),
            jax.ShapeDtypeStruct((E, D), jnp.float32),
        ],
    )(edge_attr, W_e1, b_e1, W_e2, b_e2)


def _update_body(h_ref, p_ref, wn_ref, bn_ref, o_ref):
    tot = h_ref[...] + p_ref[0] + p_ref[1]
    o_ref[...] = jax.nn.relu(
        jnp.dot(tot, wn_ref[...].T, preferred_element_type=jnp.float32)
        + bn_ref[...])


def _node_update(h, p, W_n, b_n):
    grid = (N // BN,)
    return pl.pallas_call(
        _update_body,
        grid=grid,
        in_specs=[
            pl.BlockSpec((BN, D), lambda i: (i, 0)),
            pl.BlockSpec((NC, BN, D), lambda i: (0, i, 0)),
            pl.BlockSpec((D, D), lambda i: (0, 0)),
            pl.BlockSpec((D,), lambda i: (0,)),
        ],
        out_specs=pl.BlockSpec((BN, D), lambda i: (i, 0)),
        out_shape=jax.ShapeDtypeStruct((N, D), jnp.float32),
    )(h, p, W_n, b_n)


def _final_body(h_ref, p_ref, wn_ref, bn_ref, wo_ref, bo_ref, o_ref):
    tot = h_ref[...] + p_ref[0] + p_ref[1]
    h3 = jax.nn.relu(
        jnp.dot(tot, wn_ref[...].T, preferred_element_type=jnp.float32)
        + bn_ref[...])
    logit = jnp.sum(h3 * wo_ref[...], axis=1, keepdims=True)
    o_ref[...] = jax.nn.sigmoid(logit + bo_ref[0])


def _final_head(h, p, W_n, b_n, W_out, b_out):
    grid = (N // BN,)
    return pl.pallas_call(
        _final_body,
        grid=grid,
        in_specs=[
            pl.BlockSpec((BN, D), lambda i: (i, 0)),
            pl.BlockSpec((NC, BN, D), lambda i: (0, i, 0)),
            pl.BlockSpec((D, D), lambda i: (0, 0)),
            pl.BlockSpec((D,), lambda i: (0,)),
            pl.BlockSpec((1, D), lambda i: (0, 0)),
            pl.BlockSpec((1,), lambda i: (0,)),
        ],
        out_specs=pl.BlockSpec((BN, 1), lambda i: (i, 0)),
        out_shape=jax.ShapeDtypeStruct((N, 1), jnp.float32),
    )(h, p, W_n, b_n, W_out, b_out)


def kernel(x, edge_index, edge_attr, batch, W_e1, b_e1, W_n1, b_n1,
           W_e2, b_e2, W_n2, b_n2, W_out, b_out):
    src = jnp.pad(edge_index[0].reshape(NW, NCH, CH),
                  ((0, 0), (0, NCHP - NCH), (0, 0)))
    dst = jnp.pad(edge_index[1].reshape(NW, NCH, CH),
                  ((0, 0), (0, NCHP - NCH), (0, 0)))

    emb1, emb2 = _edge_embeddings(edge_attr, W_e1, b_e1, W_e2, b_e2)

    p1 = _sc_layer(x, src, dst, emb1)
    h2 = _node_update(x, p1, W_n1, b_n1)
    p2 = _sc_layer(h2, src, dst, emb2)
    return _final_head(h2, p2, W_n2, b_n2, W_out, b_out)


# speedup vs baseline: 3.5057x; 1.1363x over previous
import functools

import jax
import jax.numpy as jnp
from jax import lax
from jax.experimental import pallas as pl
from jax.experimental.pallas import tpu as pltpu
from jax.experimental.pallas import tpu_sc as plsc

N = 10000
E = 320000
D = 128
DE = 16

NC = 2
NS = 16
NW = NC * NS
EPT = E // NW
CH = 40
NCH = EPT // CH
NP = 10240
RPT = NP // NS
ZR = 128
LANES = 16
DCH = D // LANES


NCHP = ((NCH + 7) // 8) * 8


def _sc_layer_body(h_hbm, src_hbm, dst_hbm, emb_hbm, out_hbm,
                   src0, src1, dst0, dst1, dst2, dst3, rows0, rows1,
                   emb0, emb1, zrows, aggr,
                   gsem0, gsem1, esem0, esem1, ssem0, ssem1,
                   dsem0, dsem1, dsem2, dsem3, csem0, csem1):
    c = lax.axis_index("c")
    s = lax.axis_index("s")
    wid = c * NS + s
    ebase = wid * EPT

    def zero_body(r, _):
        for j in range(DCH):
            zrows[r, pl.ds(j * LANES, LANES)] = jnp.zeros((LANES,), jnp.float32)
        return 0
    lax.fori_loop(0, ZR, zero_body, 0)
    for z in range(RPT // ZR):
        pltpu.sync_copy(zrows, aggr.at[pl.ds(s * RPT + z * ZR, ZR)])

    plsc.subcore_barrier()

    def compute_msgs(rows_c, emb_c):
        def row_body(r, _):
            for j in range(DCH):
                sl = pl.ds(j * LANES, LANES)
                v = rows_c[r, sl] + emb_c[r, sl]
                rows_c[r, sl] = jnp.maximum(v, jnp.zeros((LANES,),
                                                         jnp.float32))
            return 0
        lax.fori_loop(0, CH, row_body, 0)

    def quad_body(k, _):
        i0 = 4 * k
        hs = [pltpu.async_copy(src_hbm.at[wid, i0 + q], sb, sm)
              for q, sb, sm in ((0, src0, ssem0), (1, src1, ssem1))]
        hd = [pltpu.async_copy(dst_hbm.at[wid, i0 + q], db, dm)
              for q, db, dm in ((0, dst0, dsem0), (1, dst1, dsem1),
                                (2, dst2, dsem2), (3, dst3, dsem3))]
        hs[0].wait()
        hg0 = pltpu.async_copy(h_hbm.at[src0], rows0, gsem0)
        he0 = pltpu.async_copy(
            emb_hbm.at[pl.ds(ebase + i0 * CH, CH)], emb0, esem0)
        hs[1].wait()
        hg0.wait()
        he0.wait()
        hg1 = pltpu.async_copy(h_hbm.at[src1], rows1, gsem1)
        he1 = pltpu.async_copy(
            emb_hbm.at[pl.ds(ebase + (i0 + 1) * CH, CH)], emb1, esem1)
        hs2 = pltpu.async_copy(src_hbm.at[wid, i0 + 2], src0, ssem0)
        compute_msgs(rows0, emb0)
        hd[0].wait()
        pltpu.sync_copy(rows0, aggr.at[dst0], add=True)
        hg1.wait()
        he1.wait()
        hs3 = pltpu.async_copy(src_hbm.at[wid, i0 + 3], src1, ssem1)
        hs2.wait()
        hg2 = pltpu.async_copy(h_hbm.at[src0], rows0, gsem0)
        he2 = pltpu.async_copy(
            emb_hbm.at[pl.ds(ebase + (i0 + 2) * CH, CH)], emb0, esem0)
        compute_msgs(rows1, emb1)
        hd[1].wait()
        pltpu.sync_copy(rows1, aggr.at[dst1], add=True)
        hg2.wait()
        he2.wait()
        hs3.wait()
        hg3 = pltpu.async_copy(h_hbm.at[src1], rows1, gsem1)
        he3 = pltpu.async_copy(
            emb_hbm.at[pl.ds(ebase + (i0 + 3) * CH, CH)], emb1, esem1)
        compute_msgs(rows0, emb0)
        hd[2].wait()
        pltpu.sync_copy(rows0, aggr.at[dst2], add=True)
        hg3.wait()
        he3.wait()
        compute_msgs(rows1, emb1)
        hd[3].wait()
        pltpu.sync_copy(rows1, aggr.at[dst3], add=True)
        return 0

    def pair_body(k, _):
        i0 = 2 * k
        i1 = i0 + 1
        hs0 = pltpu.async_copy(src_hbm.at[wid, i0], src0, ssem0)
        hd0 = pltpu.async_copy(dst_hbm.at[wid, i0], dst0, dsem0)
        hs1 = pltpu.async_copy(src_hbm.at[wid, i1], src1, ssem1)
        hd1 = pltpu.async_copy(dst_hbm.at[wid, i1], dst1, dsem1)
        hs0.wait()
        hg0 = pltpu.async_copy(h_hbm.at[src0], rows0, gsem0)
        he0 = pltpu.async_copy(
            emb_hbm.at[pl.ds(ebase + i0 * CH, CH)], emb0, esem0)
        hs1.wait()
        hg0.wait()
        he0.wait()
        hg1 = pltpu.async_copy(h_hbm.at[src1], rows1, gsem1)
        he1 = pltpu.async_copy(
            emb_hbm.at[pl.ds(ebase + i1 * CH, CH)], emb1, esem1)
        compute_msgs(rows0, emb0)
        hd0.wait()
        pltpu.sync_copy(rows0, aggr.at[dst0], add=True)
        hg1.wait()
        he1.wait()
        compute_msgs(rows1, emb1)
        hd1.wait()
        pltpu.sync_copy(rows1, aggr.at[dst1], add=True)
        return 0

    NQUAD = NCH // 4
    lax.fori_loop(0, NQUAD, quad_body, 0)
    lax.fori_loop(NQUAD * 2, NCH // 2, pair_body, 0)

    plsc.subcore_barrier()
    pltpu.sync_copy(aggr.at[pl.ds(s * RPT, RPT)],
                    out_hbm.at[c, pl.ds(s * RPT, RPT)])


_sc_layer = functools.partial(
    pl.kernel,
    out_type=jax.ShapeDtypeStruct((NC, NP, D), jnp.float32),
    mesh=plsc.VectorSubcoreMesh(core_axis_name="c", subcore_axis_name="s"),
    scratch_types=[
        pltpu.VMEM((CH,), jnp.int32),
        pltpu.VMEM((CH,), jnp.int32),
        pltpu.VMEM((CH,), jnp.int32),
        pltpu.VMEM((CH,), jnp.int32),
        pltpu.VMEM((CH,), jnp.int32),
        pltpu.VMEM((CH,), jnp.int32),
        pltpu.VMEM((CH, D), jnp.float32),
        pltpu.VMEM((CH, D), jnp.float32),
        pltpu.VMEM((CH, D), jnp.float32),
        pltpu.VMEM((CH, D), jnp.float32),
        pltpu.VMEM((ZR, D), jnp.float32),
        pltpu.VMEM_SHARED((NP, D), jnp.float32),
        pltpu.SemaphoreType.DMA,
        pltpu.SemaphoreType.DMA,
        pltpu.SemaphoreType.DMA,
        pltpu.SemaphoreType.DMA,
        pltpu.SemaphoreType.DMA,
        pltpu.SemaphoreType.DMA,
        pltpu.SemaphoreType.DMA,
        pltpu.SemaphoreType.DMA,
        pltpu.SemaphoreType.DMA,
        pltpu.SemaphoreType.DMA,
        pltpu.SemaphoreType.DMA,
        pltpu.SemaphoreType.DMA,
    ],
)(_sc_layer_body)


BE = 4000
BN = 2000


def _emb_body(ea_ref, we1_ref, be1_ref, we2_ref, be2_ref, e1_ref, e2_ref):
    ea = ea_ref[...]
    e1_ref[...] = jnp.dot(ea, we1_ref[...].T,
                          preferred_element_type=jnp.float32) + be1_ref[...]
    e2_ref[...] = jnp.dot(ea, we2_ref[...].T,
                          preferred_element_type=jnp.float32) + be2_ref[...]


def _edge_embeddings(edge_attr, W_e1, b_e1, W_e2, b_e2):
    grid = (E // BE,)
    return pl.pallas_call(
        _emb_body,
        grid=grid,
        in_specs=[
            pl.BlockSpec((BE, DE), lambda i: (i, 0)),
            pl.BlockSpec((D, DE), lambda i: (0, 0)),
            pl.BlockSpec((D,), lambda i: (0,)),
            pl.BlockSpec((D, DE), lambda i: (0, 0)),
            pl.BlockSpec((D,), lambda i: (0,)),
        ],
        out_specs=[
            pl.BlockSpec((BE, D), lambda i: (i, 0)),
            pl.BlockSpec((BE, D), lambda i: (i, 0)),
        ],
        out_shape=[
            jax.ShapeDtypeStruct((E, D), jnp.float32),
            jax.ShapeDtypeStruct((E, D), jnp.float32),
        ],
    )(edge_attr, W_e1, b_e1, W_e2, b_e2)


def _update_body(h_ref, p_ref, wn_ref, bn_ref, o_ref):
    tot = h_ref[...] + p_ref[0] + p_ref[1]
    o_ref[...] = jax.nn.relu(
        jnp.dot(tot, wn_ref[...].T, preferred_element_type=jnp.float32)
        + bn_ref[...])


def _node_update(h, p, W_n, b_n):
    grid = (N // BN,)
    return pl.pallas_call(
        _update_body,
        grid=grid,
        in_specs=[
            pl.BlockSpec((BN, D), lambda i: (i, 0)),
            pl.BlockSpec((NC, BN, D), lambda i: (0, i, 0)),
            pl.BlockSpec((D, D), lambda i: (0, 0)),
            pl.BlockSpec((D,), lambda i: (0,)),
        ],
        out_specs=pl.BlockSpec((BN, D), lambda i: (i, 0)),
        out_shape=jax.ShapeDtypeStruct((N, D), jnp.float32),
    )(h, p, W_n, b_n)


def _final_body(h_ref, p_ref, wn_ref, bn_ref, wo_ref, bo_ref, o_ref):
    tot = h_ref[...] + p_ref[0] + p_ref[1]
    h3 = jax.nn.relu(
        jnp.dot(tot, wn_ref[...].T, preferred_element_type=jnp.float32)
        + bn_ref[...])
    logit = jnp.sum(h3 * wo_ref[...], axis=1, keepdims=True)
    o_ref[...] = jax.nn.sigmoid(logit + bo_ref[0])


def _final_head(h, p, W_n, b_n, W_out, b_out):
    grid = (N // BN,)
    return pl.pallas_call(
        _final_body,
        grid=grid,
        in_specs=[
            pl.BlockSpec((BN, D), lambda i: (i, 0)),
            pl.BlockSpec((NC, BN, D), lambda i: (0, i, 0)),
            pl.BlockSpec((D, D), lambda i: (0, 0)),
            pl.BlockSpec((D,), lambda i: (0,)),
            pl.BlockSpec((1, D), lambda i: (0, 0)),
            pl.BlockSpec((1,), lambda i: (0,)),
        ],
        out_specs=pl.BlockSpec((BN, 1), lambda i: (i, 0)),
        out_shape=jax.ShapeDtypeStruct((N, 1), jnp.float32),
    )(h, p, W_n, b_n, W_out, b_out)


def kernel(x, edge_index, edge_attr, batch, W_e1, b_e1, W_n1, b_n1,
           W_e2, b_e2, W_n2, b_n2, W_out, b_out):
    src = jnp.pad(edge_index[0].reshape(NW, NCH, CH),
                  ((0, 0), (0, NCHP - NCH), (0, 0)))
    dst = jnp.pad(edge_index[1].reshape(NW, NCH, CH),
                  ((0, 0), (0, NCHP - NCH), (0, 0)))

    emb1, emb2 = _edge_embeddings(edge_attr, W_e1, b_e1, W_e2, b_e2)

    p1 = _sc_layer(x, src, dst, emb1)
    h2 = _node_update(x, p1, W_n1, b_n1)
    p2 = _sc_layer(h2, src, dst, emb2)
    return _final_head(h2, p2, W_n2, b_n2, W_out, b_out)


# speedup vs baseline: 3.5136x; 1.0023x over previous
import functools

import jax
import jax.numpy as jnp
from jax import lax
from jax.experimental import pallas as pl
from jax.experimental.pallas import tpu as pltpu
from jax.experimental.pallas import tpu_sc as plsc

N = 10000
E = 320000
D = 128
DE = 16

NC = 2
NS = 16
NW = NC * NS
EPT = E // NW
CH = 40
NCH = EPT // CH
NP = 10240
RPT = NP // NS
ZR = 128
LANES = 16
DCH = D // LANES


NCHP = ((NCH + 7) // 8) * 8


def _sc_layer_body(h_hbm, idx_hbm, emb_hbm, out_hbm,
                   pk0, pk1, pk2, pk3, rows0, rows1, emb0, emb1,
                   zrows, aggr,
                   psem0, psem1, psem2, psem3, gsem0, gsem1,
                   esem0, esem1, csem0, csem1):
    c = lax.axis_index("c")
    s = lax.axis_index("s")
    wid = c * NS + s
    ebase = wid * EPT

    def zero_body(r, _):
        for j in range(DCH):
            zrows[r, pl.ds(j * LANES, LANES)] = jnp.zeros((LANES,), jnp.float32)
        return 0
    lax.fori_loop(0, ZR, zero_body, 0)
    for z in range(RPT // ZR):
        pltpu.sync_copy(zrows, aggr.at[pl.ds(s * RPT + z * ZR, ZR)])

    plsc.subcore_barrier()

    def compute_msgs(rows_c, emb_c):
        def row_body(r, _):
            for j in range(DCH):
                sl = pl.ds(j * LANES, LANES)
                v = rows_c[r, sl] + emb_c[r, sl]
                rows_c[r, sl] = jnp.maximum(v, jnp.zeros((LANES,),
                                                         jnp.float32))
            return 0
        lax.fori_loop(0, CH, row_body, 0)

    def idx_issue(i, pk, sem):
        return pltpu.async_copy(idx_hbm.at[wid, i], pk, sem)

    def gather_issue(pk, rows_b, sem):
        return pltpu.async_copy(h_hbm.at[pk.at[0]], rows_b, sem)

    def emb_issue(i, emb_b, sem):
        return pltpu.async_copy(
            emb_hbm.at[pl.ds(ebase + i * CH, CH)], emb_b, sem)

    def scat_issue(rows_b, pk, sem):
        return pltpu.async_copy(rows_b, aggr.at[pk.at[1]], sem, add=True)

    def quad_body(k, _):
        i0 = 4 * k
        hp0 = idx_issue(i0, pk0, psem0)
        hp1 = idx_issue(i0 + 1, pk1, psem1)
        hp2 = idx_issue(i0 + 2, pk2, psem2)
        hp3 = idx_issue(i0 + 3, pk3, psem3)
        hp0.wait()
        hg0 = gather_issue(pk0, rows0, gsem0)
        he0 = emb_issue(i0, emb0, esem0)
        hp1.wait()
        hg0.wait()
        he0.wait()
        hg1 = gather_issue(pk1, rows1, gsem1)
        he1 = emb_issue(i0 + 1, emb1, esem1)
        compute_msgs(rows0, emb0)
        hc0 = scat_issue(rows0, pk0, csem0)
        hp2.wait()
        hg1.wait()
        he1.wait()
        hc0.wait()
        hg2 = gather_issue(pk2, rows0, gsem0)
        he2 = emb_issue(i0 + 2, emb0, esem0)
        compute_msgs(rows1, emb1)
        hc1 = scat_issue(rows1, pk1, csem1)
        hp3.wait()
        hg2.wait()
        he2.wait()
        hc1.wait()
        hg3 = gather_issue(pk3, rows1, gsem1)
        he3 = emb_issue(i0 + 3, emb1, esem1)
        compute_msgs(rows0, emb0)
        hc2 = scat_issue(rows0, pk2, csem0)
        hg3.wait()
        he3.wait()
        compute_msgs(rows1, emb1)
        hc2.wait()
        pltpu.sync_copy(rows1, aggr.at[pk3.at[1]], add=True)
        return 0

    def pair_body(k, _):
        i0 = 2 * k
        hp0 = idx_issue(i0, pk0, psem0)
        hp1 = idx_issue(i0 + 1, pk1, psem1)
        hp0.wait()
        hg0 = gather_issue(pk0, rows0, gsem0)
        he0 = emb_issue(i0, emb0, esem0)
        hp1.wait()
        hg0.wait()
        he0.wait()
        hg1 = gather_issue(pk1, rows1, gsem1)
        he1 = emb_issue(i0 + 1, emb1, esem1)
        compute_msgs(rows0, emb0)
        pltpu.sync_copy(rows0, aggr.at[pk0.at[1]], add=True)
        hg1.wait()
        he1.wait()
        compute_msgs(rows1, emb1)
        pltpu.sync_copy(rows1, aggr.at[pk1.at[1]], add=True)
        return 0

    NQUAD = NCH // 4
    lax.fori_loop(0, NQUAD, quad_body, 0)
    lax.fori_loop(NQUAD * 2, NCH // 2, pair_body, 0)

    plsc.subcore_barrier()
    pltpu.sync_copy(aggr.at[pl.ds(s * RPT, RPT)],
                    out_hbm.at[c, pl.ds(s * RPT, RPT)])


_sc_layer = functools.partial(
    pl.kernel,
    out_type=jax.ShapeDtypeStruct((NC, NP, D), jnp.float32),
    mesh=plsc.VectorSubcoreMesh(core_axis_name="c", subcore_axis_name="s"),
    scratch_types=[
        pltpu.VMEM((2, CH), jnp.int32),
        pltpu.VMEM((2, CH), jnp.int32),
        pltpu.VMEM((2, CH), jnp.int32),
        pltpu.VMEM((2, CH), jnp.int32),
        pltpu.VMEM((CH, D), jnp.float32),
        pltpu.VMEM((CH, D), jnp.float32),
        pltpu.VMEM((CH, D), jnp.float32),
        pltpu.VMEM((CH, D), jnp.float32),
        pltpu.VMEM((ZR, D), jnp.float32),
        pltpu.VMEM_SHARED((NP, D), jnp.float32),
        pltpu.SemaphoreType.DMA,
        pltpu.SemaphoreType.DMA,
        pltpu.SemaphoreType.DMA,
        pltpu.SemaphoreType.DMA,
        pltpu.SemaphoreType.DMA,
        pltpu.SemaphoreType.DMA,
        pltpu.SemaphoreType.DMA,
        pltpu.SemaphoreType.DMA,
        pltpu.SemaphoreType.DMA,
        pltpu.SemaphoreType.DMA,
    ],
)(_sc_layer_body)


BE = 4000
BN = 2000


def _emb_body(ea_ref, we_ref, be_ref, e_ref):
    e_ref[...] = jnp.dot(ea_ref[...], we_ref[...].T,
                         preferred_element_type=jnp.float32) + be_ref[...]


def _edge_embeddings(edge_attr, W_e, b_e):
    grid = (E // BE,)
    return pl.pallas_call(
        _emb_body,
        grid=grid,
        in_specs=[
            pl.BlockSpec((BE, DE), lambda i: (i, 0)),
            pl.BlockSpec((D, DE), lambda i: (0, 0)),
            pl.BlockSpec((D,), lambda i: (0,)),
        ],
        out_specs=pl.BlockSpec((BE, D), lambda i: (i, 0)),
        out_shape=jax.ShapeDtypeStruct((E, D), jnp.float32),
    )(edge_attr, W_e, b_e)


def _update_body(h_ref, p_ref, wn_ref, bn_ref, o_ref):
    tot = h_ref[...] + p_ref[0] + p_ref[1]
    o_ref[...] = jax.nn.relu(
        jnp.dot(tot, wn_ref[...].T, preferred_element_type=jnp.float32)
        + bn_ref[...])


def _node_update(h, p, W_n, b_n):
    grid = (N // BN,)
    return pl.pallas_call(
        _update_body,
        grid=grid,
        in_specs=[
            pl.BlockSpec((BN, D), lambda i: (i, 0)),
            pl.BlockSpec((NC, BN, D), lambda i: (0, i, 0)),
            pl.BlockSpec((D, D), lambda i: (0, 0)),
            pl.BlockSpec((D,), lambda i: (0,)),
        ],
        out_specs=pl.BlockSpec((BN, D), lambda i: (i, 0)),
        out_shape=jax.ShapeDtypeStruct((N, D), jnp.float32),
    )(h, p, W_n, b_n)


def _final_body(h_ref, p_ref, wn_ref, bn_ref, wo_ref, bo_ref, o_ref):
    tot = h_ref[...] + p_ref[0] + p_ref[1]
    h3 = jax.nn.relu(
        jnp.dot(tot, wn_ref[...].T, preferred_element_type=jnp.float32)
        + bn_ref[...])
    logit = jnp.sum(h3 * wo_ref[...], axis=1, keepdims=True)
    o_ref[...] = jax.nn.sigmoid(logit + bo_ref[0])


def _final_head(h, p, W_n, b_n, W_out, b_out):
    grid = (N // BN,)
    return pl.pallas_call(
        _final_body,
        grid=grid,
        in_specs=[
            pl.BlockSpec((BN, D), lambda i: (i, 0)),
            pl.BlockSpec((NC, BN, D), lambda i: (0, i, 0)),
            pl.BlockSpec((D, D), lambda i: (0, 0)),
            pl.BlockSpec((D,), lambda i: (0,)),
            pl.BlockSpec((1, D), lambda i: (0, 0)),
            pl.BlockSpec((1,), lambda i: (0,)),
        ],
        out_specs=pl.BlockSpec((BN, 1), lambda i: (i, 0)),
        out_shape=jax.ShapeDtypeStruct((N, 1), jnp.float32),
    )(h, p, W_n, b_n, W_out, b_out)


def kernel(x, edge_index, edge_attr, batch, W_e1, b_e1, W_n1, b_n1,
           W_e2, b_e2, W_n2, b_n2, W_out, b_out):
    idx = jnp.pad(edge_index.reshape(2, NW, NCH, CH).transpose(1, 2, 0, 3),
                  ((0, 0), (0, NCHP - NCH), (0, 0), (0, 0)))

    emb1 = _edge_embeddings(edge_attr, W_e1, b_e1)
    emb2 = _edge_embeddings(edge_attr, W_e2, b_e2)

    p1 = _sc_layer(x, idx, emb1)
    h2 = _node_update(x, p1, W_n1, b_n1)
    p2 = _sc_layer(h2, idx, emb2)
    return _final_head(h2, p2, W_n2, b_n2, W_out, b_out)


# speedup vs baseline: 4.0503x; 1.1527x over previous
import functools

import jax
import jax.numpy as jnp
from jax import lax
from jax.experimental import pallas as pl
from jax.experimental.pallas import tpu as pltpu
from jax.experimental.pallas import tpu_sc as plsc

N = 10000
E = 320000
D = 128
DE = 16

NC = 2
NS = 16
NW = NC * NS
EPT = E // NW
CH = 40
NCH = EPT // CH
NP = 10240
RPT = NP // NS
ZR = 128
LANES = 16
DCH = D // LANES


NCHP = ((NCH + 7) // 8) * 8


def _sc_layer_body(h_hbm, idx_hbm, emb_hbm, out_hbm,
                   pk0, pk1, pk2, pk3, pk4, pk5,
                   rows0, rows1, rows2, emb0, emb1, emb2,
                   zrows, aggr,
                   psem0, psem1, psem2, psem3, psem4, psem5,
                   gsem0, gsem1, gsem2, esem0, esem1, esem2,
                   csem0, csem1, csem2):
    c = lax.axis_index("c")
    s = lax.axis_index("s")
    wid = c * NS + s
    ebase = wid * EPT

    def zero_body(r, _):
        for j in range(DCH):
            zrows[r, pl.ds(j * LANES, LANES)] = jnp.zeros((LANES,), jnp.float32)
        return 0
    lax.fori_loop(0, ZR, zero_body, 0)
    for z in range(RPT // ZR):
        pltpu.sync_copy(zrows, aggr.at[pl.ds(s * RPT + z * ZR, ZR)])

    plsc.subcore_barrier()

    def compute_msgs(rows_c, emb_c):
        def row_body(r, _):
            for j in range(DCH):
                sl = pl.ds(j * LANES, LANES)
                v = rows_c[r, sl] + emb_c[r, sl]
                rows_c[r, sl] = jnp.maximum(v, jnp.zeros((LANES,),
                                                         jnp.float32))
            return 0
        lax.fori_loop(0, CH, row_body, 0)

    def idx_issue(i, pk, sem):
        return pltpu.async_copy(idx_hbm.at[wid, i], pk, sem)

    def gather_issue(pk, rows_b, sem):
        return pltpu.async_copy(h_hbm.at[pk.at[0]], rows_b, sem)

    def emb_issue(i, emb_b, sem):
        return pltpu.async_copy(
            emb_hbm.at[pl.ds(ebase + i * CH, CH)], emb_b, sem)

    def scat_issue(rows_b, pk, sem):
        return pltpu.async_copy(rows_b, aggr.at[pk.at[1]], sem, add=True)

    def sext_body(k, _):
        i0 = 6 * k
        hp = [idx_issue(i0 + q, pkq, psq)
              for q, pkq, psq in ((0, pk0, psem0), (1, pk1, psem1),
                                  (2, pk2, psem2), (3, pk3, psem3),
                                  (4, pk4, psem4), (5, pk5, psem5))]
        hp[0].wait()
        hg0 = gather_issue(pk0, rows0, gsem0)
        he0 = emb_issue(i0, emb0, esem0)
        hp[1].wait()
        hg1 = gather_issue(pk1, rows1, gsem1)
        he1 = emb_issue(i0 + 1, emb1, esem1)
        hp[2].wait()
        hg2 = gather_issue(pk2, rows2, gsem2)
        he2 = emb_issue(i0 + 2, emb2, esem2)
        hg0.wait()
        he0.wait()
        compute_msgs(rows0, emb0)
        hc0 = scat_issue(rows0, pk0, csem0)
        hp[3].wait()
        hc0.wait()
        hg3 = gather_issue(pk3, rows0, gsem0)
        he3 = emb_issue(i0 + 3, emb0, esem0)
        hg1.wait()
        he1.wait()
        compute_msgs(rows1, emb1)
        hc1 = scat_issue(rows1, pk1, csem1)
        hp[4].wait()
        hc1.wait()
        hg4 = gather_issue(pk4, rows1, gsem1)
        he4 = emb_issue(i0 + 4, emb1, esem1)
        hg2.wait()
        he2.wait()
        compute_msgs(rows2, emb2)
        hc2 = scat_issue(rows2, pk2, csem2)
        hp[5].wait()
        hc2.wait()
        hg5 = gather_issue(pk5, rows2, gsem2)
        he5 = emb_issue(i0 + 5, emb2, esem2)
        hg3.wait()
        he3.wait()
        compute_msgs(rows0, emb0)
        hc3 = scat_issue(rows0, pk3, csem0)
        hg4.wait()
        he4.wait()
        compute_msgs(rows1, emb1)
        hc4 = scat_issue(rows1, pk4, csem1)
        hc3.wait()
        hg5.wait()
        he5.wait()
        compute_msgs(rows2, emb2)
        hc4.wait()
        pltpu.sync_copy(rows2, aggr.at[pk5.at[1]], add=True)
        return 0

    NSEXT = NCH // 6
    lax.fori_loop(0, NSEXT, sext_body, 0)

    TAIL0 = NSEXT * 6
    NTAIL = NCH - TAIL0
    slots = ((pk0, rows0, emb0, psem0, gsem0, esem0, csem0),
             (pk1, rows1, emb1, psem1, gsem1, esem1, csem1),
             (pk2, rows2, emb2, psem2, gsem2, esem2, csem2))
    for base in range(TAIL0, NCH, 3):
        m = min(3, NCH - base)
        hts = [idx_issue(base + j, slots[j][0], slots[j][3])
               for j in range(m)]
        hgs = []
        for j in range(m):
            pkq, rq, eq, _, gsq, esq, _ = slots[j]
            hts[j].wait()
            hgs.append((gather_issue(pkq, rq, gsq),
                        emb_issue(base + j, eq, esq)))
        for j in range(m):
            pkq, rq, eq, _, _, _, _ = slots[j]
            hgs[j][0].wait()
            hgs[j][1].wait()
            compute_msgs(rq, eq)
            pltpu.sync_copy(rq, aggr.at[pkq.at[1]], add=True)

    plsc.subcore_barrier()
    pltpu.sync_copy(aggr.at[pl.ds(s * RPT, RPT)],
                    out_hbm.at[c, pl.ds(s * RPT, RPT)])


_sc_layer = functools.partial(
    pl.kernel,
    out_type=jax.ShapeDtypeStruct((NC, NP, D), jnp.float32),
    mesh=plsc.VectorSubcoreMesh(core_axis_name="c", subcore_axis_name="s"),
    scratch_types=[
        pltpu.VMEM((2, CH), jnp.int32),
        pltpu.VMEM((2, CH), jnp.int32),
        pltpu.VMEM((2, CH), jnp.int32),
        pltpu.VMEM((2, CH), jnp.int32),
        pltpu.VMEM((2, CH), jnp.int32),
        pltpu.VMEM((2, CH), jnp.int32),
        pltpu.VMEM((CH, D), jnp.float32),
        pltpu.VMEM((CH, D), jnp.float32),
        pltpu.VMEM((CH, D), jnp.float32),
        pltpu.VMEM((CH, D), jnp.float32),
        pltpu.VMEM((CH, D), jnp.float32),
        pltpu.VMEM((CH, D), jnp.float32),
        pltpu.VMEM((ZR, D), jnp.float32),
        pltpu.VMEM_SHARED((NP, D), jnp.float32),
    ] + [pltpu.SemaphoreType.DMA] * 15,
)(_sc_layer_body)


BE = 4000
BN = 2000


def _emb_body(ea_ref, we_ref, be_ref, e_ref):
    e_ref[...] = jnp.dot(ea_ref[...], we_ref[...].T,
                         preferred_element_type=jnp.float32) + be_ref[...]


def _edge_embeddings(edge_attr, W_e, b_e):
    grid = (E // BE,)
    return pl.pallas_call(
        _emb_body,
        grid=grid,
        in_specs=[
            pl.BlockSpec((BE, DE), lambda i: (i, 0)),
            pl.BlockSpec((D, DE), lambda i: (0, 0)),
            pl.BlockSpec((D,), lambda i: (0,)),
        ],
        out_specs=pl.BlockSpec((BE, D), lambda i: (i, 0)),
        out_shape=jax.ShapeDtypeStruct((E, D), jnp.float32),
    )(edge_attr, W_e, b_e)


def _update_body(h_ref, p_ref, wn_ref, bn_ref, o_ref):
    tot = h_ref[...] + p_ref[0] + p_ref[1]
    o_ref[...] = jax.nn.relu(
        jnp.dot(tot, wn_ref[...].T, preferred_element_type=jnp.float32)
        + bn_ref[...])


def _node_update(h, p, W_n, b_n):
    grid = (N // BN,)
    return pl.pallas_call(
        _update_body,
        grid=grid,
        in_specs=[
            pl.BlockSpec((BN, D), lambda i: (i, 0)),
            pl.BlockSpec((NC, BN, D), lambda i: (0, i, 0)),
            pl.BlockSpec((D, D), lambda i: (0, 0)),
            pl.BlockSpec((D,), lambda i: (0,)),
        ],
        out_specs=pl.BlockSpec((BN, D), lambda i: (i, 0)),
        out_shape=jax.ShapeDtypeStruct((N, D), jnp.float32),
    )(h, p, W_n, b_n)


def _final_body(h_ref, p_ref, wn_ref, bn_ref, wo_ref, bo_ref, o_ref):
    tot = h_ref[...] + p_ref[0] + p_ref[1]
    h3 = jax.nn.relu(
        jnp.dot(tot, wn_ref[...].T, preferred_element_type=jnp.float32)
        + bn_ref[...])
    logit = jnp.sum(h3 * wo_ref[...], axis=1, keepdims=True)
    o_ref[...] = jax.nn.sigmoid(logit + bo_ref[0])


def _final_head(h, p, W_n, b_n, W_out, b_out):
    grid = (N // BN,)
    return pl.pallas_call(
        _final_body,
        grid=grid,
        in_specs=[
            pl.BlockSpec((BN, D), lambda i: (i, 0)),
            pl.BlockSpec((NC, BN, D), lambda i: (0, i, 0)),
            pl.BlockSpec((D, D), lambda i: (0, 0)),
            pl.BlockSpec((D,), lambda i: (0,)),
            pl.BlockSpec((1, D), lambda i: (0, 0)),
            pl.BlockSpec((1,), lambda i: (0,)),
        ],
        out_specs=pl.BlockSpec((BN, 1), lambda i: (i, 0)),
        out_shape=jax.ShapeDtypeStruct((N, 1), jnp.float32),
    )(h, p, W_n, b_n, W_out, b_out)


def kernel(x, edge_index, edge_attr, batch, W_e1, b_e1, W_n1, b_n1,
           W_e2, b_e2, W_n2, b_n2, W_out, b_out):
    idx = jnp.pad(edge_index.reshape(2, NW, NCH, CH).transpose(1, 2, 0, 3),
                  ((0, 0), (0, NCHP - NCH), (0, 0), (0, 0)))

    emb1 = _edge_embeddings(edge_attr, W_e1, b_e1)
    emb2 = _edge_embeddings(edge_attr, W_e2, b_e2)

    p1 = _sc_layer(x, idx, emb1)
    h2 = _node_update(x, p1, W_n1, b_n1)
    p2 = _sc_layer(h2, idx, emb2)
    return _final_head(h2, p2, W_n2, b_n2, W_out, b_out)
